# baseline (speedup 1.0000x reference)
import jax
import jax.numpy as jnp
from jax.experimental import pallas as pl
from jax.experimental.pallas import tpu as pltpu

N = 1024
D = 128
H = 4
C = 32
HID = H * C
DLIN = 1024
DOUT = 256
TJ = 128
NT = N // TJ


def _gat_layer(xin, Wl, Wr, attF, w4P, adjT_ref, h_scr, xr_scr):
    xl = jnp.dot(xin, Wl, preferred_element_type=jnp.float32)
    xr = jnp.dot(xin, Wr, preferred_element_type=jnp.float32)
    xr_scr[...] = xr * attF
    xlpT = (xl * attF).T

    def tile_body(t, carry):
        row0 = t * TJ
        m = adjT_ref[pl.ds(row0, TJ), :]
        mneg = (m - 1.0) * jnp.float32(1e30)
        for h in range(H):
            xl_h = xl[:, h * C:(h + 1) * C]
            xlpT_h = xlpT[h * C:(h + 1) * C, :]
            al6 = 0.6 * jnp.sum(xlpT_h, axis=0, keepdims=True)
            xrp_blk = xr_scr[pl.ds(row0, TJ), h * C:(h + 1) * C]
            ar6 = 0.6 * jnp.sum(xrp_blk, axis=1, keepdims=True)
            xrp_cj = xrp_blk.T
            z = xrp_cj[:, :, None] + xlpT_h[:, None, :]
            w4 = w4P[:, h][:, None, None]
            s = jnp.sum(jnp.abs(z) * w4, axis=0) + (al6 + ar6)
            sm = s + mneg
            rmax = jnp.max(sm, axis=1, keepdims=True)
            rmax = jnp.where(rmax > jnp.float32(-5e29), rmax, 0.0)
            e = jnp.exp(sm - rmax)
            den = jnp.sum(e, axis=1, keepdims=True)
            o = jnp.dot(e, xl_h, preferred_element_type=jnp.float32) \
                / (den + 1e-16)
            h_scr[pl.ds(row0, TJ), h * C:(h + 1) * C] = o
        return carry

    jax.lax.fori_loop(0, NT, tile_body, 0)
    return h_scr[...]


def _batchnorm_relu(hb, g, be):
    mu = jnp.mean(hb, axis=0, keepdims=True)
    d = hb - mu
    var = jnp.mean(d * d, axis=0, keepdims=True)
    hn = d / jnp.sqrt(var + 1e-5) * g + be
    return jnp.maximum(hn, 0.0)


def _body(x_ref, adjT_ref,
          Wl0_ref, Wr0_ref, attF0_ref, w4P0_ref, b0_ref,
          Wl1_ref, Wr1_ref, attF1_ref, w4P1_ref, b1_ref,
          Wl2_ref, Wr2_ref, attF2_ref, w4P2_ref, b2_ref,
          g0_ref, be0_ref, g1_ref, be1_ref,
          lw1_ref, lb1_ref, lw2_ref, lb2_ref,
          out_ref, h0_scr, h1_scr, h2_scr, xr_scr):
    x = x_ref[...]

    h = _gat_layer(x, Wl0_ref[...], Wr0_ref[...], attF0_ref[...],
                   w4P0_ref[...], adjT_ref, h0_scr, xr_scr) + b0_ref[...]
    h = _batchnorm_relu(h, g0_ref[...], be0_ref[...])

    h = _gat_layer(h, Wl1_ref[...], Wr1_ref[...], attF1_ref[...],
                   w4P1_ref[...], adjT_ref, h1_scr, xr_scr) + b1_ref[...]
    h = _batchnorm_relu(h, g1_ref[...], be1_ref[...])

    h = _gat_layer(h, Wl2_ref[...], Wr2_ref[...], attF2_ref[...],
                   w4P2_ref[...], adjT_ref, h2_scr, xr_scr) + b2_ref[...]

    a1 = jnp.dot(h, lw1_ref[...], preferred_element_type=jnp.float32)
    a1 = jnp.maximum(a1 + lb1_ref[...], 0.0)
    out_ref[...] = jnp.dot(a1, lw2_ref[...],
                           preferred_element_type=jnp.float32) + lb2_ref[...]


def _build(interpret=False):
    return pl.pallas_call(
        _body,
        out_shape=jax.ShapeDtypeStruct((N, DOUT), jnp.float32),
        scratch_shapes=[pltpu.VMEM((N, HID), jnp.float32)] * 4,
        compiler_params=pltpu.CompilerParams(
            vmem_limit_bytes=120 * 1024 * 1024),
        interpret=interpret,
    )


def kernel(x, adj, batch, Wl0, Wr0, att0, b0, Wl1, Wr1, att1, b1,
           Wl2, Wr2, att2, b2, g0, be0, g1, be1, lw1, lb1, lw2, lb2):
    adjT = adj.T
    call = _build()

    def prep(att):
        return att.reshape(1, HID), (0.4 * jnp.sign(att)).T

    attF0, w4P0 = prep(att0)
    attF1, w4P1 = prep(att1)
    attF2, w4P2 = prep(att2)
    return call(
        x, adjT,
        Wl0, Wr0, attF0, w4P0, b0.reshape(1, HID),
        Wl1, Wr1, attF1, w4P1, b1.reshape(1, HID),
        Wl2, Wr2, attF2, w4P2, b2.reshape(1, HID),
        g0.reshape(1, HID), be0.reshape(1, HID),
        g1.reshape(1, HID), be1.reshape(1, HID),
        lw1, lb1.reshape(1, DLIN), lw2, lb2.reshape(1, DOUT))


# speedup vs baseline: 1.0553x; 1.0553x over previous
import jax
import jax.numpy as jnp
from jax.experimental import pallas as pl
from jax.experimental.pallas import tpu as pltpu

N = 1024
D = 128
H = 4
C = 32
HID = H * C
DLIN = 1024
DOUT = 256
TJ = 256
NT = N // TJ


def _gat_layer(xin, Wl, Wr, attF, w4P, adjT_ref, h_scr, xr_scr):
    xl = jnp.dot(xin, Wl, preferred_element_type=jnp.float32)
    xr = jnp.dot(xin, Wr, preferred_element_type=jnp.float32)
    xr_scr[...] = xr * attF
    xlpT = (xl * attF).T

    def tile_body(t, carry):
        row0 = t * TJ
        m = adjT_ref[pl.ds(row0, TJ), :]
        edge = m != 0.0
        for h in range(H):
            xl_h = xl[:, h * C:(h + 1) * C]
            xlpT_h = xlpT[h * C:(h + 1) * C, :]
            al6 = 0.6 * jnp.sum(xlpT_h, axis=0, keepdims=True)
            xrp_blk = xr_scr[pl.ds(row0, TJ), h * C:(h + 1) * C]
            ar6 = 0.6 * jnp.sum(xrp_blk, axis=1, keepdims=True)
            xrp_cj = xrp_blk.T
            z = xrp_cj[:, :, None] + xlpT_h[:, None, :]
            w4 = w4P[:, h][:, None, None]
            s = jnp.sum(jnp.abs(z) * w4, axis=0) + (al6 + ar6)
            neg = jnp.float32(-1e30)
            sm = jnp.where(edge, s, neg)
            rmax = jnp.max(sm, axis=1, keepdims=True)
            rmax = jnp.where(rmax > 0.5 * neg, rmax, 0.0)
            e = jnp.exp(sm - rmax)
            den = jnp.sum(e, axis=1, keepdims=True)
            p = e / (den + 1e-16)
            o = jnp.dot(p, xl_h, preferred_element_type=jnp.float32)
            h_scr[pl.ds(row0, TJ), h * C:(h + 1) * C] = o
        return carry

    jax.lax.fori_loop(0, NT, tile_body, 0)
    return h_scr[...]


def _batchnorm_relu(hb, g, be):
    mu = jnp.mean(hb, axis=0, keepdims=True)
    d = hb - mu
    var = jnp.mean(d * d, axis=0, keepdims=True)
    hn = d / jnp.sqrt(var + 1e-5) * g + be
    return jnp.maximum(hn, 0.0)


def _body(x_ref, adjT_ref,
          Wl0_ref, Wr0_ref, attF0_ref, w4P0_ref, b0_ref,
          Wl1_ref, Wr1_ref, attF1_ref, w4P1_ref, b1_ref,
          Wl2_ref, Wr2_ref, attF2_ref, w4P2_ref, b2_ref,
          g0_ref, be0_ref, g1_ref, be1_ref,
          lw1_ref, lb1_ref, lw2_ref, lb2_ref,
          out_ref, h0_scr, h1_scr, h2_scr, xr_scr):
    x = x_ref[...]

    h = _gat_layer(x, Wl0_ref[...], Wr0_ref[...], attF0_ref[...],
                   w4P0_ref[...], adjT_ref, h0_scr, xr_scr) + b0_ref[...]
    h = _batchnorm_relu(h, g0_ref[...], be0_ref[...])

    h = _gat_layer(h, Wl1_ref[...], Wr1_ref[...], attF1_ref[...],
                   w4P1_ref[...], adjT_ref, h1_scr, xr_scr) + b1_ref[...]
    h = _batchnorm_relu(h, g1_ref[...], be1_ref[...])

    h = _gat_layer(h, Wl2_ref[...], Wr2_ref[...], attF2_ref[...],
                   w4P2_ref[...], adjT_ref, h2_scr, xr_scr) + b2_ref[...]

    a1 = jnp.dot(h, lw1_ref[...], preferred_element_type=jnp.float32)
    a1 = jnp.maximum(a1 + lb1_ref[...], 0.0)
    out_ref[...] = jnp.dot(a1, lw2_ref[...],
                           preferred_element_type=jnp.float32) + lb2_ref[...]


def _build(interpret=False):
    return pl.pallas_call(
        _body,
        out_shape=jax.ShapeDtypeStruct((N, DOUT), jnp.float32),
        scratch_shapes=[pltpu.VMEM((N, HID), jnp.float32)] * 4,
        compiler_params=pltpu.CompilerParams(
            vmem_limit_bytes=120 * 1024 * 1024),
        interpret=interpret,
    )


def kernel(x, adj, batch, Wl0, Wr0, att0, b0, Wl1, Wr1, att1, b1,
           Wl2, Wr2, att2, b2, g0, be0, g1, be1, lw1, lb1, lw2, lb2):
    adjT = adj.T
    call = _build()

    def prep(att):
        return att.reshape(1, HID), (0.4 * jnp.sign(att)).T

    attF0, w4P0 = prep(att0)
    attF1, w4P1 = prep(att1)
    attF2, w4P2 = prep(att2)
    return call(
        x, adjT,
        Wl0, Wr0, attF0, w4P0, b0.reshape(1, HID),
        Wl1, Wr1, attF1, w4P1, b1.reshape(1, HID),
        Wl2, Wr2, attF2, w4P2, b2.reshape(1, HID),
        g0.reshape(1, HID), be0.reshape(1, HID),
        g1.reshape(1, HID), be1.reshape(1, HID),
        lw1, lb1.reshape(1, DLIN), lw2, lb2.reshape(1, DOUT))


# speedup vs baseline: 1.0956x; 1.0382x over previous
import jax
import jax.numpy as jnp
from jax.experimental import pallas as pl
from jax.experimental.pallas import tpu as pltpu

N = 1024
D = 128
H = 4
C = 32
HID = H * C
DLIN = 1024
DOUT = 256
TJ = 64
NT = N // TJ


def _gat_layer(xin, Wl, Wr, attF, w4P, adjT_ref, h_scr, xr_scr):
    xl = jnp.dot(xin, Wl, preferred_element_type=jnp.float32)
    xr = jnp.dot(xin, Wr, preferred_element_type=jnp.float32)
    xr_scr[...] = xr * attF
    xlpT = (xl * attF).T

    def tile_body(t, carry):
        row0 = t * TJ
        m = adjT_ref[pl.ds(row0, TJ), :]
        edge = m != 0.0
        for h in range(H):
            xl_h = xl[:, h * C:(h + 1) * C]
            xlpT_h = xlpT[h * C:(h + 1) * C, :]
            al6 = 0.6 * jnp.sum(xlpT_h, axis=0, keepdims=True)
            xrp_blk = xr_scr[pl.ds(row0, TJ), h * C:(h + 1) * C]
            ar6 = 0.6 * jnp.sum(xrp_blk, axis=1, keepdims=True)
            xrp_cj = xrp_blk.T
            z = xrp_cj[:, :, None] + xlpT_h[:, None, :]
            w4 = w4P[:, h][:, None, None]
            s = jnp.sum(jnp.abs(z) * w4, axis=0) + (al6 + ar6)
            neg = jnp.float32(-1e30)
            sm = jnp.where(edge, s, neg)
            rmax = jnp.max(sm, axis=1, keepdims=True)
            rmax = jnp.where(rmax > 0.5 * neg, rmax, 0.0)
            e = jnp.exp(sm - rmax)
            den = jnp.sum(e, axis=1, keepdims=True)
            p = e / (den + 1e-16)
            o = jnp.dot(p, xl_h, preferred_element_type=jnp.float32)
            h_scr[pl.ds(row0, TJ), h * C:(h + 1) * C] = o
        return carry

    jax.lax.fori_loop(0, NT, tile_body, 0)
    return h_scr[...]


def _batchnorm_relu(hb, g, be):
    mu = jnp.mean(hb, axis=0, keepdims=True)
    d = hb - mu
    var = jnp.mean(d * d, axis=0, keepdims=True)
    hn = d / jnp.sqrt(var + 1e-5) * g + be
    return jnp.maximum(hn, 0.0)


def _body(x_ref, adjT_ref,
          Wl0_ref, Wr0_ref, attF0_ref, w4P0_ref, b0_ref,
          Wl1_ref, Wr1_ref, attF1_ref, w4P1_ref, b1_ref,
          Wl2_ref, Wr2_ref, attF2_ref, w4P2_ref, b2_ref,
          g0_ref, be0_ref, g1_ref, be1_ref,
          lw1_ref, lb1_ref, lw2_ref, lb2_ref,
          out_ref, h0_scr, h1_scr, h2_scr, xr_scr):
    x = x_ref[...]

    h = _gat_layer(x, Wl0_ref[...], Wr0_ref[...], attF0_ref[...],
                   w4P0_ref[...], adjT_ref, h0_scr, xr_scr) + b0_ref[...]
    h = _batchnorm_relu(h, g0_ref[...], be0_ref[...])

    h = _gat_layer(h, Wl1_ref[...], Wr1_ref[...], attF1_ref[...],
                   w4P1_ref[...], adjT_ref, h1_scr, xr_scr) + b1_ref[...]
    h = _batchnorm_relu(h, g1_ref[...], be1_ref[...])

    h = _gat_layer(h, Wl2_ref[...], Wr2_ref[...], attF2_ref[...],
                   w4P2_ref[...], adjT_ref, h2_scr, xr_scr) + b2_ref[...]

    a1 = jnp.dot(h, lw1_ref[...], preferred_element_type=jnp.float32)
    a1 = jnp.maximum(a1 + lb1_ref[...], 0.0)
    out_ref[...] = jnp.dot(a1, lw2_ref[...],
                           preferred_element_type=jnp.float32) + lb2_ref[...]


def _build(interpret=False):
    return pl.pallas_call(
        _body,
        out_shape=jax.ShapeDtypeStruct((N, DOUT), jnp.float32),
        scratch_shapes=[pltpu.VMEM((N, HID), jnp.float32)] * 4,
        compiler_params=pltpu.CompilerParams(
            vmem_limit_bytes=120 * 1024 * 1024),
        interpret=interpret,
    )


def kernel(x, adj, batch, Wl0, Wr0, att0, b0, Wl1, Wr1, att1, b1,
           Wl2, Wr2, att2, b2, g0, be0, g1, be1, lw1, lb1, lw2, lb2):
    adjT = adj.T
    call = _build()

    def prep(att):
        return att.reshape(1, HID), (0.4 * jnp.sign(att)).T

    attF0, w4P0 = prep(att0)
    attF1, w4P1 = prep(att1)
    attF2, w4P2 = prep(att2)
    return call(
        x, adjT,
        Wl0, Wr0, attF0, w4P0, b0.reshape(1, HID),
        Wl1, Wr1, attF1, w4P1, b1.reshape(1, HID),
        Wl2, Wr2, attF2, w4P2, b2.reshape(1, HID),
        g0.reshape(1, HID), be0.reshape(1, HID),
        g1.reshape(1, HID), be1.reshape(1, HID),
        lw1, lb1.reshape(1, DLIN), lw2, lb2.reshape(1, DOUT))
